# baseline (speedup 1.0000x reference)
import functools

import jax
import jax.numpy as jnp
from jax import lax
from jax.experimental import pallas as pl
from jax.experimental.pallas import tpu as pltpu
from jax.experimental.pallas import tpu_sc as plsc

N_FACTORS = 32
N_ROWS = 1000000
SUP = 128
RPS = SUP // N_FACTORS
BATCH = 16384
NC = 2
NS = 16
NW = NC * NS
BPW = BATCH // NW
CHUNK = 128
NCH = BPW // CHUNK
LANES = 16


def _mf_body(user_r, item_r, uf_r, if_r, out_r,
             uidx, iidx, usup, isup, urows, irows, outv, sem):
    wid = lax.axis_index("s") * NC + lax.axis_index("c")

    pltpu.sync_copy(user_r.at[wid], uidx)
    pltpu.sync_copy(item_r.at[wid], iidx)

    def xform(t, c):
        j = t // (CHUNK // LANES)
        o = (t % (CHUNK // LANES)) * LANES
        usup[j, pl.ds(o, LANES)] = lax.shift_right_logical(uidx[j, pl.ds(o, LANES)], 2)
        isup[j, pl.ds(o, LANES)] = lax.shift_right_logical(iidx[j, pl.ds(o, LANES)], 2)
        return c

    lax.fori_loop(0, BPW // LANES, xform, 0)

    lane = lax.iota(jnp.int32, LANES)

    for j in range(NCH):
        cu = pltpu.async_copy(uf_r.at[usup.at[j]], urows, sem)
        ci = pltpu.async_copy(if_r.at[isup.at[j]], irows, sem)
        cu.wait()
        ci.wait()

        def group(gg, c):
            o = gg * LANES
            co_u = (uidx[j, pl.ds(o, LANES)] & (RPS - 1)) * N_FACTORS
            co_i = (iidx[j, pl.ds(o, LANES)] & (RPS - 1)) * N_FACTORS
            acc = jnp.zeros((LANES,), jnp.float32)
            for r in range(LANES):
                cou = co_u[r]
                coi = co_i[r]
                row = o + r
                u0 = urows[row, pl.ds(cou, LANES)]
                u1 = urows[row, pl.ds(cou + LANES, LANES)]
                v0 = irows[row, pl.ds(coi, LANES)]
                v1 = irows[row, pl.ds(coi + LANES, LANES)]
                tot = jnp.sum(u0 * v0 + u1 * v1)
                acc = jnp.where(lane == r, tot, acc)
            outv[pl.ds(j * CHUNK + o, LANES)] = acc
            return c

        lax.fori_loop(0, CHUNK // LANES, group, 0)

    pltpu.sync_copy(outv, out_r.at[wid])


_mf = functools.partial(
    pl.kernel,
    mesh=plsc.VectorSubcoreMesh(core_axis_name="c", subcore_axis_name="s"),
    out_type=jax.ShapeDtypeStruct((NW, BPW), jnp.float32),
    scratch_types=[
        pltpu.VMEM((NCH, CHUNK), jnp.int32),
        pltpu.VMEM((NCH, CHUNK), jnp.int32),
        pltpu.VMEM((NCH, CHUNK), jnp.int32),
        pltpu.VMEM((NCH, CHUNK), jnp.int32),
        pltpu.VMEM((CHUNK, SUP), jnp.float32),
        pltpu.VMEM((CHUNK, SUP), jnp.float32),
        pltpu.VMEM((BPW,), jnp.float32),
        pltpu.SemaphoreType.DMA,
    ],
    compiler_params=pltpu.CompilerParams(needs_layout_passes=False),
)(_mf_body)


def kernel(user, item, user_factors, item_factors):
    u = user.astype(jnp.int32).reshape(NW, NCH, CHUNK)
    i = item.astype(jnp.int32).reshape(NW, NCH, CHUNK)
    uf = user_factors.reshape(N_ROWS // RPS, SUP)
    itf = item_factors.reshape(N_ROWS // RPS, SUP)
    out = _mf(u, i, uf, itf)
    return out.reshape(BATCH)


# speedup vs baseline: 1.5083x; 1.5083x over previous
import functools

import jax
import jax.numpy as jnp
from jax import lax
from jax.experimental import pallas as pl
from jax.experimental.pallas import tpu as pltpu
from jax.experimental.pallas import tpu_sc as plsc

N_FACTORS = 32
BATCH = 16384
NC = 2
NS = 16
NW = NC * NS
BPW = BATCH // NW
CHUNK = 128
NCH = BPW // CHUNK
LANES = 16


def _mf_body(user_r, item_r, uf_r, if_r, out_r,
             uidx, iidx, urows, irows, outv, sem):
    wid = lax.axis_index("s") * NC + lax.axis_index("c")

    pltpu.sync_copy(user_r.at[wid], uidx)
    pltpu.sync_copy(item_r.at[wid], iidx)

    lane = lax.iota(jnp.int32, LANES)

    for j in range(NCH):
        def issue(p0, c):
            uv = uidx[j, pl.ds(p0 * LANES, LANES)]
            iv = iidx[j, pl.ds(p0 * LANES, LANES)]
            for q in range(LANES):
                p = p0 * LANES + q
                pltpu.async_copy(uf_r.at[pl.ds(uv[q], 1)],
                                 urows.at[pl.ds(p, 1)], sem)
                pltpu.async_copy(if_r.at[pl.ds(iv[q], 1)],
                                 irows.at[pl.ds(p, 1)], sem)
            return c

        lax.fori_loop(0, CHUNK // LANES, issue, 0)

        pltpu.make_async_copy(uf_r.at[pl.ds(0, CHUNK)], urows, sem).wait()
        pltpu.make_async_copy(if_r.at[pl.ds(0, CHUNK)], irows, sem).wait()

        def group(gg, c):
            o = gg * LANES
            acc = jnp.zeros((LANES,), jnp.float32)
            for r in range(LANES):
                row = o + r
                s0 = urows[row, pl.ds(0, LANES)] * irows[row, pl.ds(0, LANES)]
                s1 = urows[row, pl.ds(LANES, LANES)] * irows[row, pl.ds(LANES, LANES)]
                tot = jnp.sum(s0 + s1)
                acc = jnp.where(lane == r, tot, acc)
            outv[pl.ds(j * CHUNK + o, LANES)] = acc
            return c

        lax.fori_loop(0, CHUNK // LANES, group, 0)

    pltpu.sync_copy(outv, out_r.at[wid])


_mf = functools.partial(
    pl.kernel,
    mesh=plsc.VectorSubcoreMesh(core_axis_name="c", subcore_axis_name="s"),
    out_type=jax.ShapeDtypeStruct((NW, BPW), jnp.float32),
    scratch_types=[
        pltpu.VMEM((NCH, CHUNK), jnp.int32),
        pltpu.VMEM((NCH, CHUNK), jnp.int32),
        pltpu.VMEM((CHUNK, N_FACTORS), jnp.float32),
        pltpu.VMEM((CHUNK, N_FACTORS), jnp.float32),
        pltpu.VMEM((BPW,), jnp.float32),
        pltpu.SemaphoreType.DMA,
    ],
    compiler_params=pltpu.CompilerParams(needs_layout_passes=False),
)(_mf_body)


def kernel(user, item, user_factors, item_factors):
    u = user.astype(jnp.int32).reshape(NW, NCH, CHUNK)
    i = item.astype(jnp.int32).reshape(NW, NCH, CHUNK)
    out = _mf(u, i, user_factors, item_factors)
    return out.reshape(BATCH)
